# baseline (speedup 1.0000x reference)
import functools

import jax
import jax.numpy as jnp
from jax import lax
from jax.experimental import pallas as pl
from jax.experimental.pallas import tpu as pltpu
from jax.experimental.pallas import tpu_sc as plsc

_N = 10000
_D = 128
_E = 320000
_NC = 2
_NS = 16
_NW = _NC * _NS
_EPT = _E // _NW
_CH = 80
_NCH = _EPT // _CH
_NSEG = 5
_SCH = _NCH // _NSEG
_NP = 10240
_RPT = _NP // _NS
_ZR = 64
_PAIRS = (_SCH - 1) // 2


def _spmm(gnnp, src3, dst3, w3, zrows):
  mesh = plsc.VectorSubcoreMesh(core_axis_name="c", subcore_axis_name="s")

  @functools.partial(
      pl.kernel,
      out_type=jax.ShapeDtypeStruct((_NC, _NP, _D), jnp.float32),
      mesh=mesh,
      scratch_types=[
          pltpu.VMEM_SHARED((_NP, _D), jnp.float32),
          pltpu.VMEM((_SCH, _CH), jnp.int32),
          pltpu.VMEM((_SCH, _CH), jnp.int32),
          pltpu.VMEM((_SCH, _CH), jnp.float32),
          pltpu.VMEM((_CH, _D // 2), jnp.float32),
          pltpu.VMEM((_CH, _D // 2), jnp.float32),
          pltpu.VMEM((_CH, _D // 2), jnp.float32),
          pltpu.VMEM((_CH, _D), jnp.float32),
          pltpu.SemaphoreType.DMA,
          pltpu.SemaphoreType.DMA,
          pltpu.SemaphoreType.DMA,
      ],
      compiler_params=pltpu.CompilerParams(needs_layout_passes=False,
                                           use_tc_tiling_on_sc=False),
  )
  def spmm(gnn_hbm, src_hbm, dst_hbm, w_hbm, z_hbm, out_hbm,
           accum, src_v, dst_v, w_v, prow0, prow1, prow2, full_v,
           sem0, sem1, sem2):
    cid = lax.axis_index("c")
    sid = lax.axis_index("s")
    wid = sid * _NC + cid

    row0 = sid * _RPT
    for kk in range(_RPT // _ZR):
      pltpu.sync_copy(z_hbm, accum.at[pl.ds(row0 + kk * _ZR, _ZR)])
    plsc.subcore_barrier()

    def _expand_scale(prow, g):
      def _grp(t, c2):
        base = pl.multiple_of(t * 16, 16)
        wv16 = w_v[g, pl.ds(base, 16)]
        for u in range(16):
          e = base + u
          ws = wv16[u]
          for a in range(_D // 32):
            pair = plsc.bitcast(prow[e, pl.ds(a * 16, 16)], jnp.bfloat16)
            lo, hi = plsc.unpack(pair, format=plsc.PackFormat.INTERLEAVED)
            full_v[e, pl.ds(a * 32, 16)] = lo * ws
            full_v[e, pl.ds(a * 32 + 16, 16)] = hi * ws
        return c2

      lax.fori_loop(0, _CH // 16, _grp, 0)

    def _step(g, prow, sem, fire_ahead):
      pltpu.make_async_copy(gnn_hbm.at[src_v.at[g]], prow, sem).wait()
      _expand_scale(prow, g)
      if fire_ahead:
        @pl.when(g + 3 < _SCH)
        def _():
          pltpu.async_copy(gnn_hbm.at[src_v.at[g + 3]], prow, sem)
      pltpu.sync_copy(full_v, accum.at[dst_v.at[g]], add=True)

    def _seg(s, c0):
      pltpu.sync_copy(src_hbm.at[wid, s], src_v)
      pltpu.sync_copy(dst_hbm.at[wid, s], dst_v)
      pltpu.sync_copy(w_hbm.at[wid, s], w_v)
      pltpu.async_copy(gnn_hbm.at[src_v.at[0]], prow0, sem0)
      pltpu.async_copy(gnn_hbm.at[src_v.at[1]], prow1, sem1)
      pltpu.async_copy(gnn_hbm.at[src_v.at[2]], prow2, sem2)

      def _triple(q, c):
        g = 3 * q
        _step(g, prow0, sem0, True)
        _step(g + 1, prow1, sem1, True)
        _step(g + 2, prow2, sem2, True)
        return c

      lax.fori_loop(0, _SCH // 3, _triple, 0)
      _step(_SCH - 1, prow0, sem0, False)
      return c0

    lax.fori_loop(0, _NSEG, _seg, 0)
    plsc.subcore_barrier()

    for kk in range(_RPT // _CH):
      r = row0 + kk * _CH
      pltpu.sync_copy(accum.at[pl.ds(r, _CH)], full_v)
      pltpu.sync_copy(full_v, out_hbm.at[cid, pl.ds(r, _CH)])

  return spmm(gnnp, src3, dst3, w3, zrows)


def _pack_words(x):
  outs = []
  for a in range(_D // 32):
    lo = x[:, 32 * a:32 * a + 16]
    hi = x[:, 32 * a + 16:32 * a + 32]
    lob = lax.bitcast_convert_type(lo.astype(jnp.bfloat16),
                                   jnp.uint16).astype(jnp.uint32)
    hib = lax.bitcast_convert_type(hi.astype(jnp.bfloat16),
                                   jnp.uint16).astype(jnp.uint32)
    outs.append(lax.bitcast_convert_type((hib << 16) | lob, jnp.float32))
  return jnp.concatenate(outs, axis=1)


def _packf(x):
  blk = 1000

  def body(x_ref, o_ref):
    o_ref[...] = _pack_words(x_ref[...])

  return pl.pallas_call(
      body,
      out_shape=jax.ShapeDtypeStruct((_N, _D // 2), jnp.float32),
      grid=(_N // blk,),
      in_specs=[pl.BlockSpec((blk, _D), lambda i: (i, 0))],
      out_specs=pl.BlockSpec((blk, _D // 2), lambda i: (i, 0)),
  )(x)


def _pairpack(p):
  blk = 640

  def body(p_ref, o_ref):
    o_ref[...] = _pack_words(p_ref[0] + p_ref[1])

  return pl.pallas_call(
      body,
      out_shape=jax.ShapeDtypeStruct((_NP, _D // 2), jnp.float32),
      grid=(_NP // blk,),
      in_specs=[pl.BlockSpec((2, blk, _D), lambda i: (0, i, 0))],
      out_specs=pl.BlockSpec((blk, _D // 2), lambda i: (i, 0)),
  )(p)


def _head(p, W_emb, b_emb, W_lft, b_lft, W_rgt, b_rgt):
  blk = 1000
  dn = (((1,), (1,)), ((), ()))

  def body(p_ref, we_ref, be_ref, wl_ref, bl_ref, wr_ref, br_ref,
           emb_ref, lft_ref, rgt_ref):
    x = p_ref[0] + p_ref[1]
    e = lax.dot_general(x, we_ref[...], dn,
                        preferred_element_type=jnp.float32) + be_ref[...]
    nrm = jnp.sqrt(jnp.sum(e * e, axis=1, keepdims=True))
    emb_ref[...] = e / jnp.maximum(nrm, 1e-12)
    l = jnp.tanh(lax.dot_general(x, wl_ref[0], dn,
                                 preferred_element_type=jnp.float32)
                 + bl_ref[0]) + x
    lft_ref[...] = jnp.tanh(lax.dot_general(l, wl_ref[1], dn,
                                            preferred_element_type=jnp.float32)
                            + bl_ref[1])
    r = jnp.tanh(lax.dot_general(x, wr_ref[0], dn,
                                 preferred_element_type=jnp.float32)
                 + br_ref[0]) + x
    rgt_ref[...] = jnp.tanh(lax.dot_general(r, wr_ref[1], dn,
                                            preferred_element_type=jnp.float32)
                            + br_ref[1])

  out_sds = jax.ShapeDtypeStruct((_N, _D), jnp.float32)
  return pl.pallas_call(
      body,
      out_shape=(out_sds, out_sds, out_sds),
      grid=(_N // blk,),
      in_specs=[
          pl.BlockSpec((2, blk, _D), lambda i: (0, i, 0)),
          pl.BlockSpec((_D, _D), lambda i: (0, 0)),
          pl.BlockSpec((1, _D), lambda i: (0, 0)),
          pl.BlockSpec((2, _D, _D), lambda i: (0, 0, 0)),
          pl.BlockSpec((2, 1, _D), lambda i: (0, 0, 0)),
          pl.BlockSpec((2, _D, _D), lambda i: (0, 0, 0)),
          pl.BlockSpec((2, 1, _D), lambda i: (0, 0, 0)),
      ],
      out_specs=(
          pl.BlockSpec((blk, _D), lambda i: (i, 0)),
          pl.BlockSpec((blk, _D), lambda i: (i, 0)),
          pl.BlockSpec((blk, _D), lambda i: (i, 0)),
      ),
  )(p, W_emb, b_emb, W_lft, b_lft, W_rgt, b_rgt)


def kernel(feat, edge_index, edge_weight, W_fr, b_fr, W_emb, b_emb,
           W_lft, b_lft, W_rgt, b_rgt):
  src3 = edge_index[0].reshape(_NW, _NSEG, _SCH, _CH)
  dst3 = edge_index[1].reshape(_NW, _NSEG, _SCH, _CH)
  w3 = edge_weight.reshape(_NW, _NSEG, _SCH, _CH)
  zrows = jnp.zeros((_ZR, _D), jnp.float32)
  p1 = _spmm(_packf(feat), src3, dst3, w3, zrows)
  p2 = _spmm(_pairpack(p1), src3, dst3, w3, zrows)
  return _head(p2, W_emb, b_emb.reshape(1, _D), W_lft,
               b_lft.reshape(2, 1, _D), W_rgt, b_rgt.reshape(2, 1, _D))


# speedup vs baseline: 1.9464x; 1.9464x over previous
import functools

import jax
import jax.numpy as jnp
from jax import lax
from jax.experimental import pallas as pl
from jax.experimental.pallas import tpu as pltpu
from jax.experimental.pallas import tpu_sc as plsc

_N = 10000
_D = 128
_E = 320000
_NC = 2
_NS = 16
_NW = _NC * _NS
_EPT = _E // _NW
_CH = 80
_NCH = _EPT // _CH
_NSEG = 5
_SCH = _NCH // _NSEG
_NP = 10240
_RPT = _NP // _NS
_ZR = 64
_PAIRS = (_SCH - 1) // 2


def _spmm(gnn, src3, dst3, w3, zrows):
  mesh = plsc.VectorSubcoreMesh(core_axis_name="c", subcore_axis_name="s")

  @functools.partial(
      pl.kernel,
      out_type=jax.ShapeDtypeStruct((_NC, _NP, _D), jnp.float32),
      mesh=mesh,
      scratch_types=[
          pltpu.VMEM_SHARED((_NP, _D), jnp.float32),
          pltpu.VMEM((_SCH, _CH), jnp.int32),
          pltpu.VMEM((_SCH, _CH), jnp.int32),
          pltpu.VMEM((_SCH, _CH), jnp.float32),
          pltpu.VMEM((_CH, _D), jnp.float32),
          pltpu.VMEM((_CH, _D), jnp.float32),
          pltpu.VMEM((_CH, _D), jnp.float32),
          pltpu.SemaphoreType.DMA,
          pltpu.SemaphoreType.DMA,
          pltpu.SemaphoreType.DMA,
      ],
  )
  def spmm(gnn_hbm, src_hbm, dst_hbm, w_hbm, z_hbm, out_hbm,
           accum, src_v, dst_v, w_v, rows0, rows1, rows2,
           sem0, sem1, sem2):
    cid = lax.axis_index("c")
    sid = lax.axis_index("s")
    wid = sid * _NC + cid

    row0 = sid * _RPT
    for kk in range(_RPT // _ZR):
      pltpu.sync_copy(z_hbm, accum.at[pl.ds(row0 + kk * _ZR, _ZR)])
    plsc.subcore_barrier()

    def _scale(rows, g):
      def _grp(t, c2):
        base = pl.multiple_of(t * 16, 16)
        wv16 = w_v[g, pl.ds(base, 16)]
        for u in range(16):
          e = base + u
          ws = wv16[u]
          for j in range(_D // 16):
            rows[e, pl.ds(j * 16, 16)] = rows[e, pl.ds(j * 16, 16)] * ws
        return c2

      lax.fori_loop(0, _CH // 16, _grp, 0)

    def _step(g, rows, sem, fire_ahead):
      pltpu.make_async_copy(gnn_hbm.at[src_v.at[g]], rows, sem).wait()
      _scale(rows, g)
      pltpu.sync_copy(rows, accum.at[dst_v.at[g]], add=True)
      if fire_ahead:
        @pl.when(g + 3 < _SCH)
        def _():
          pltpu.async_copy(gnn_hbm.at[src_v.at[g + 3]], rows, sem)

    def _seg(s, c0):
      pltpu.sync_copy(src_hbm.at[wid, s], src_v)
      pltpu.sync_copy(dst_hbm.at[wid, s], dst_v)
      pltpu.sync_copy(w_hbm.at[wid, s], w_v)
      pltpu.async_copy(gnn_hbm.at[src_v.at[0]], rows0, sem0)
      pltpu.async_copy(gnn_hbm.at[src_v.at[1]], rows1, sem1)
      pltpu.async_copy(gnn_hbm.at[src_v.at[2]], rows2, sem2)

      def _triple(q, c):
        g = 3 * q
        _step(g, rows0, sem0, True)
        _step(g + 1, rows1, sem1, True)
        _step(g + 2, rows2, sem2, True)
        return c

      lax.fori_loop(0, _SCH // 3, _triple, 0)
      _step(_SCH - 1, rows0, sem0, False)
      return c0

    lax.fori_loop(0, _NSEG, _seg, 0)
    plsc.subcore_barrier()

    for kk in range(_RPT // _CH):
      r = row0 + kk * _CH
      pltpu.sync_copy(accum.at[pl.ds(r, _CH)], rows0)
      pltpu.sync_copy(rows0, out_hbm.at[cid, pl.ds(r, _CH)])

  return spmm(gnn, src3, dst3, w3, zrows)


def _pairsum(p):
  blk = 640

  def body(p_ref, o_ref):
    o_ref[...] = p_ref[0] + p_ref[1]

  return pl.pallas_call(
      body,
      out_shape=jax.ShapeDtypeStruct((_NP, _D), jnp.float32),
      grid=(_NP // blk,),
      in_specs=[pl.BlockSpec((2, blk, _D), lambda i: (0, i, 0))],
      out_specs=pl.BlockSpec((blk, _D), lambda i: (i, 0)),
  )(p)


def _head(p, W_emb, b_emb, W_lft, b_lft, W_rgt, b_rgt):
  blk = 1000
  dn = (((1,), (1,)), ((), ()))

  def body(p_ref, we_ref, be_ref, wl_ref, bl_ref, wr_ref, br_ref,
           emb_ref, lft_ref, rgt_ref):
    x = p_ref[0] + p_ref[1]
    e = lax.dot_general(x, we_ref[...], dn,
                        preferred_element_type=jnp.float32) + be_ref[...]
    nrm = jnp.sqrt(jnp.sum(e * e, axis=1, keepdims=True))
    emb_ref[...] = e / jnp.maximum(nrm, 1e-12)
    l = jnp.tanh(lax.dot_general(x, wl_ref[0], dn,
                                 preferred_element_type=jnp.float32)
                 + bl_ref[0]) + x
    lft_ref[...] = jnp.tanh(lax.dot_general(l, wl_ref[1], dn,
                                            preferred_element_type=jnp.float32)
                            + bl_ref[1])
    r = jnp.tanh(lax.dot_general(x, wr_ref[0], dn,
                                 preferred_element_type=jnp.float32)
                 + br_ref[0]) + x
    rgt_ref[...] = jnp.tanh(lax.dot_general(r, wr_ref[1], dn,
                                            preferred_element_type=jnp.float32)
                            + br_ref[1])

  out_sds = jax.ShapeDtypeStruct((_N, _D), jnp.float32)
  return pl.pallas_call(
      body,
      out_shape=(out_sds, out_sds, out_sds),
      grid=(_N // blk,),
      in_specs=[
          pl.BlockSpec((2, blk, _D), lambda i: (0, i, 0)),
          pl.BlockSpec((_D, _D), lambda i: (0, 0)),
          pl.BlockSpec((1, _D), lambda i: (0, 0)),
          pl.BlockSpec((2, _D, _D), lambda i: (0, 0, 0)),
          pl.BlockSpec((2, 1, _D), lambda i: (0, 0, 0)),
          pl.BlockSpec((2, _D, _D), lambda i: (0, 0, 0)),
          pl.BlockSpec((2, 1, _D), lambda i: (0, 0, 0)),
      ],
      out_specs=(
          pl.BlockSpec((blk, _D), lambda i: (i, 0)),
          pl.BlockSpec((blk, _D), lambda i: (i, 0)),
          pl.BlockSpec((blk, _D), lambda i: (i, 0)),
      ),
  )(p, W_emb, b_emb, W_lft, b_lft, W_rgt, b_rgt)


def kernel(feat, edge_index, edge_weight, W_fr, b_fr, W_emb, b_emb,
           W_lft, b_lft, W_rgt, b_rgt):
  src3 = edge_index[0].reshape(_NW, _NSEG, _SCH, _CH)
  dst3 = edge_index[1].reshape(_NW, _NSEG, _SCH, _CH)
  w3 = edge_weight.reshape(_NW, _NSEG, _SCH, _CH)
  zrows = jnp.zeros((_ZR, _D), jnp.float32)
  p1 = _spmm(feat, src3, dst3, w3, zrows)
  gnn1 = _pairsum(p1)
  p2 = _spmm(gnn1, src3, dst3, w3, zrows)
  return _head(p2, W_emb, b_emb.reshape(1, _D), W_lft,
               b_lft.reshape(2, 1, _D), W_rgt, b_rgt.reshape(2, 1, _D))
